# baseline (speedup 1.0000x reference)
import functools

import jax
import jax.numpy as jnp
from jax import lax
from jax.experimental import pallas as pl
from jax.experimental.pallas import tpu as pltpu
from jax.experimental.pallas import tpu_sc as plsc

B = 16384 * 26
D = 32
NC = 2
NS = 16
NW = NC * NS
NU = 26 * 128
U_PER_W = NU // NW
NSLOT = 4


def _sc_gather_t(table, idx_t):
    mesh = plsc.VectorSubcoreMesh(core_axis_name="c", subcore_axis_name="s")

    @functools.partial(
        pl.kernel,
        mesh=mesh,
        out_type=jax.ShapeDtypeStruct((26, 4, 128, 1024), jnp.float32),
        compiler_params=pltpu.CompilerParams(
            use_tc_tiling_on_sc=False, needs_layout_passes=False),
        scratch_types=(
            [pltpu.VMEM((U_PER_W, 128), jnp.int32)]
            + [pltpu.VMEM((128, D), jnp.float32) for _ in range(NSLOT)]
            + [pltpu.VMEM((4096,), jnp.float32) for _ in range(NSLOT)]
            + [pltpu.SemaphoreType.DMA for _ in range(2 * NSLOT)]
        ),
    )
    def k(table_hbm, idx_hbm, out_hbm, *scratch):
        idx_v = scratch[0]
        rows = scratch[1 : 1 + NSLOT]
        tbuf = scratch[1 + NSLOT : 1 + 2 * NSLOT]
        gsem = scratch[1 + 2 * NSLOT : 1 + 3 * NSLOT]
        ssem = scratch[1 + 3 * NSLOT : 1 + 4 * NSLOT]

        wid = lax.axis_index("s") * NC + lax.axis_index("c")
        ubase = wid * U_PER_W
        pltpu.sync_copy(idx_hbm.at[pl.ds(ubase, U_PER_W)], idx_v)

        def unit_ct(n):
            u = ubase + n
            return lax.div(u, 128), lax.rem(u, 128)

        def start_gather(n, b):
            pltpu.async_copy(table_hbm.at[idx_v.at[n]], rows[b], gsem[b])

        def wait_gather(n, b):
            pltpu.make_async_copy(
                table_hbm.at[idx_v.at[n]], rows[b], gsem[b]).wait()

        def start_write(n, b):
            c, tc = unit_ct(n)
            for tr in range(4):
                pltpu.async_copy(
                    tbuf[b].at[pl.ds(tr * 1024, 1024)],
                    out_hbm.at[c, tr, tc], ssem[b])

        def wait_write(n, b):
            c, tc = unit_ct(n)
            for tr in range(4):
                pltpu.make_async_copy(
                    tbuf[b].at[pl.ds(tr * 1024, 1024)],
                    out_hbm.at[c, tr, tc], ssem[b]).wait()

        row_ids = [lax.iota(jnp.int32, 16) + 16 * h for h in range(8)]
        cols = [jnp.full((16,), j, jnp.int32) for j in range(D)]

        for b in range(NSLOT):
            start_gather(b, b)

        @pl.loop(0, U_PER_W, step=NSLOT)
        def _(n):
            for b in range(NSLOT):
                nb = n + b
                wait_gather(nb, b)
                @pl.when(nb >= NSLOT)
                def _():
                    wait_write(nb - NSLOT, b)
                for j in range(D):
                    for h in range(8):
                        tbuf[b][pl.ds(j * 128 + 16 * h, 16)] = (
                            plsc.load_gather(rows[b], [row_ids[h], cols[j]]))
                start_write(nb, b)
                @pl.when(nb + NSLOT < U_PER_W)
                def _():
                    start_gather(nb + NSLOT, b)

        for b in range(NSLOT):
            wait_write(U_PER_W - NSLOT + b, b)

    return k(table, idx_t)


def kernel(entity_indices, entity_embeddings):
    idx_t = entity_indices.T.reshape(NU, 128)
    out = _sc_gather_t(entity_embeddings, idx_t)
    return (out.reshape(26, 4, 128, 8, 128)
            .transpose(2, 4, 0, 1, 3)
            .reshape(16384, 26, D))


# speedup vs baseline: 1.1534x; 1.1534x over previous
import functools

import jax
import jax.numpy as jnp
from jax import lax
from jax.experimental import pallas as pl
from jax.experimental.pallas import tpu as pltpu
from jax.experimental.pallas import tpu_sc as plsc

V = 1_000_000
D = 32
NC = 2
NS = 16
NW = NC * NS

KT = 2
CE = KT * 128
NFULL = 7812
NCHUNK_A = NFULL // KT
TAIL_E = V - NFULL * 128

NU = 26 * 128
U_PER_W = NU // NW
NSLOT = 4


def _relayout(table_t, tail_flat):
    mesh = plsc.VectorSubcoreMesh(core_axis_name="c", subcore_axis_name="s")

    @functools.partial(
        pl.kernel,
        mesh=mesh,
        out_type=jax.ShapeDtypeStruct((V * D,), jnp.float32),
        compiler_params=pltpu.CompilerParams(needs_layout_passes=False),
        scratch_types=(
            [pltpu.VMEM((8, CE), jnp.float32) for _ in range(8)]
            + [pltpu.VMEM((CE * D,), jnp.float32) for _ in range(2)]
            + [pltpu.SemaphoreType.DMA for _ in range(4)]
        ),
    )
    def k(tab_hbm, tail_hbm, out_hbm, *scratch):
        strips = [scratch[0:4], scratch[4:8]]
        rows = scratch[8:10]
        gsem = scratch[10:12]
        ssem = scratch[12:14]

        wid = lax.axis_index("s") * NC + lax.axis_index("c")
        cstart = 122 * wid + 2 * jnp.minimum(wid, 1)
        nch = 122 + 2 * jnp.where(wid == 0, 1, 0)

        def start_reads(n, b):
            e0 = (cstart + n) * CE
            for tr in range(4):
                pltpu.async_copy(
                    tab_hbm.at[pl.ds(8 * tr, 8), pl.ds(e0, CE)],
                    strips[b][tr], gsem[b])

        def wait_reads(n, b):
            e0 = (cstart + n) * CE
            for tr in range(4):
                pltpu.make_async_copy(
                    tab_hbm.at[pl.ds(8 * tr, 8), pl.ds(e0, CE)],
                    strips[b][tr], gsem[b]).wait()

        def start_write(n, b):
            e0 = (cstart + n) * CE
            pltpu.async_copy(rows[b], out_hbm.at[pl.ds(e0 * D, CE * D)],
                             ssem[b])

        def wait_write(n, b):
            e0 = (cstart + n) * CE
            pltpu.make_async_copy(
                rows[b], out_hbm.at[pl.ds(e0 * D, CE * D)], ssem[b]).wait()

        ibase = lax.iota(jnp.int32, 16) * D

        def transpose_block(b, t, l0):
            x0 = t * 128 + 16 * l0
            vals = [strips[b][c // 8][c % 8, pl.ds(x0, 16)] for c in range(D)]
            idxs = [ibase + (x0 * D + c) for c in range(D)]
            for c in range(D):
                plsc.store_scatter(rows[b], [idxs[c]], vals[c])

        start_reads(0, 0)
        start_reads(1, 1)

        @pl.loop(0, nch, step=2)
        def _(ch):
            for b in range(2):
                n = ch + b
                wait_reads(n, b)
                @pl.when(n >= 2)
                def _():
                    wait_write(n - 2, b)
                for t in range(KT):
                    for l0 in range(8):
                        transpose_block(b, t, l0)
                start_write(n, b)
                @pl.when(n + 2 < nch)
                def _():
                    start_reads(n + 2, b)

        wait_write(nch - 2, 0)
        wait_write(nch - 1, 1)

        @pl.when(wid == 0)
        def _():
            pltpu.sync_copy(tail_hbm, rows[0].at[pl.ds(0, TAIL_E * D)])
            pltpu.sync_copy(rows[0].at[pl.ds(0, TAIL_E * D)],
                            out_hbm.at[pl.ds(NFULL * 128 * D, TAIL_E * D)])

    return k(table_t, tail_flat)


def _gather_t(table_r, idx_t):
    mesh = plsc.VectorSubcoreMesh(core_axis_name="c", subcore_axis_name="s")

    @functools.partial(
        pl.kernel,
        mesh=mesh,
        out_type=jax.ShapeDtypeStruct((26, 4, 128, 1024), jnp.float32),
        compiler_params=pltpu.CompilerParams(
            use_tc_tiling_on_sc=False, needs_layout_passes=False),
        scratch_types=(
            [pltpu.VMEM((U_PER_W, 128), jnp.int32)]
            + [pltpu.VMEM((128, D), jnp.float32) for _ in range(NSLOT)]
            + [pltpu.VMEM((4096,), jnp.float32) for _ in range(NSLOT)]
            + [pltpu.SemaphoreType.DMA for _ in range(2 * NSLOT)]
        ),
    )
    def k(table_hbm, idx_hbm, out_hbm, *scratch):
        idx_v = scratch[0]
        rows = scratch[1 : 1 + NSLOT]
        tbuf = scratch[1 + NSLOT : 1 + 2 * NSLOT]
        gsem = scratch[1 + 2 * NSLOT : 1 + 3 * NSLOT]
        ssem = scratch[1 + 3 * NSLOT : 1 + 4 * NSLOT]

        wid = lax.axis_index("s") * NC + lax.axis_index("c")
        ubase = wid * U_PER_W
        pltpu.sync_copy(idx_hbm.at[pl.ds(ubase, U_PER_W)], idx_v)

        def unit_ct(n):
            u = ubase + n
            return lax.div(u, 128), lax.rem(u, 128)

        def start_gather(n, b):
            pltpu.async_copy(table_hbm.at[idx_v.at[n]], rows[b], gsem[b])

        def wait_gather(n, b):
            pltpu.make_async_copy(
                table_hbm.at[idx_v.at[n]], rows[b], gsem[b]).wait()

        def start_write(n, b):
            c, tc = unit_ct(n)
            for tr in range(4):
                pltpu.async_copy(
                    tbuf[b].at[pl.ds(tr * 1024, 1024)],
                    out_hbm.at[c, tr, tc], ssem[b])

        def wait_write(n, b):
            c, tc = unit_ct(n)
            for tr in range(4):
                pltpu.make_async_copy(
                    tbuf[b].at[pl.ds(tr * 1024, 1024)],
                    out_hbm.at[c, tr, tc], ssem[b]).wait()

        row_ids = [lax.iota(jnp.int32, 16) + 16 * h for h in range(8)]
        cols = [jnp.full((16,), j, jnp.int32) for j in range(D)]

        for b in range(NSLOT):
            start_gather(b, b)

        @pl.loop(0, U_PER_W, step=NSLOT)
        def _(n):
            for b in range(NSLOT):
                nb = n + b
                wait_gather(nb, b)
                @pl.when(nb >= NSLOT)
                def _():
                    wait_write(nb - NSLOT, b)
                for j2 in range(0, D, 2):
                    vals = [plsc.load_gather(rows[b], [row_ids[h], cols[j]])
                            for j in (j2, j2 + 1) for h in range(8)]
                    i = 0
                    for j in (j2, j2 + 1):
                        for h in range(8):
                            tbuf[b][pl.ds(j * 128 + 16 * h, 16)] = vals[i]
                            i += 1
                start_write(nb, b)
                @pl.when(nb + NSLOT < U_PER_W)
                def _():
                    start_gather(nb + NSLOT, b)

        for b in range(NSLOT):
            wait_write(U_PER_W - NSLOT + b, b)

    return k(table_r, idx_t)


def kernel(entity_indices, entity_embeddings):
    tail_flat = entity_embeddings[NFULL * 128 :].reshape(TAIL_E * D)
    table_r = _relayout(entity_embeddings.T, tail_flat).reshape(V, D)
    idx_t = entity_indices.T.reshape(NU, 128)
    out = _gather_t(table_r, idx_t)
    return (out.reshape(26, 4, 128, 8, 128)
            .transpose(2, 4, 0, 1, 3)
            .reshape(16384, 26, D))
